# baseline (speedup 1.0000x reference)
import functools

import jax
import jax.numpy as jnp
from jax import lax
from jax.experimental import pallas as pl
from jax.experimental.pallas import tpu as pltpu
from jax.experimental.pallas import tpu_sc as plsc

VOCAB = 16
D = 1024
TOKENS = 4 * 8192

_info = plsc.get_sparse_core_info()
NC, NS = _info.num_cores, _info.num_subcores
NW = NC * NS
TPW = TOKENS // NW
CHUNK = 16
ROWS = 2 * CHUNK
NCHUNK = TPW // CHUNK


def _build_t2_body(w_ref, cm_ref, t2_ref):
    w = w_ref[...]
    cm = cm_ref[...]
    onehot = (cm == lax.broadcasted_iota(jnp.int32, (VOCAB, VOCAB), 1))
    sel = jax.lax.dot(onehot.astype(jnp.float32), w,
                      precision=jax.lax.Precision.HIGHEST)
    revp = (lax.broadcasted_iota(jnp.int32, (D, D), 0)
            + lax.broadcasted_iota(jnp.int32, (D, D), 1)) == (D - 1)
    t2_ref[0:VOCAB, :] = w
    t2_ref[VOCAB:2 * VOCAB, :] = jax.lax.dot(
        sel, revp.astype(jnp.float32), precision=jax.lax.Precision.HIGHEST)


def _build_t2(weight, comp_map):
    return pl.pallas_call(
        _build_t2_body,
        out_shape=jax.ShapeDtypeStruct((2 * VOCAB, D), jnp.float32),
    )(weight, comp_map.reshape(VOCAB, 1))


def _vgather(v, idx):
    return lax.gather(
        v, idx[:, None],
        dimension_numbers=lax.GatherDimensionNumbers(
            offset_dims=(), collapsed_slice_dims=(0,), start_index_map=(0,)),
        slice_sizes=(1,),
        mode=lax.GatherScatterMode.PROMISE_IN_BOUNDS)


def _sc_gather_body(ids_hbm, t2_hbm, out_hbm, ids_v, idx_v, buf, semg, semw):
    wid = lax.axis_index("s") * NC + lax.axis_index("c")
    base = wid * TPW
    pltpu.sync_copy(ids_hbm.at[pl.ds(base, TPW)], ids_v)

    lane = lax.iota(jnp.int32, 16)
    half = lax.shift_right_logical(lane, 1)
    offs = VOCAB * lax.rem(lane, 2)

    def build_idx(m, carry):
        v = ids_v[pl.ds(m * 16, 16)]
        w0 = _vgather(v, half) + offs
        w1 = _vgather(v, half + 8) + offs
        idx_v[pl.ds(m * 32, 16)] = w0
        idx_v[pl.ds(m * 32 + 16, 16)] = w1
        return carry

    lax.fori_loop(0, TPW // 16, build_idx, 0)

    def idx_at(i):
        return idx_v.at[pl.ds(i * ROWS, ROWS)]

    def out_at(i):
        return out_hbm.at[pl.ds(2 * base + i * ROWS, ROWS)]

    pltpu.async_copy(t2_hbm.at[idx_at(0)], buf.at[0], semg.at[0])

    def step(i, carry):
        b = lax.rem(i, 2)

        @pl.when(i >= 1)
        def _():
            pltpu.make_async_copy(buf.at[1 - b], out_at(i - 1),
                                  semw.at[1 - b]).wait()

        @pl.when(i + 1 < NCHUNK)
        def _():
            pltpu.async_copy(t2_hbm.at[idx_at(i + 1)], buf.at[1 - b],
                             semg.at[1 - b])

        pltpu.make_async_copy(t2_hbm.at[idx_at(i)], buf.at[b],
                              semg.at[b]).wait()
        pltpu.async_copy(buf.at[b], out_at(i), semw.at[b])
        return carry

    lax.fori_loop(0, NCHUNK, step, 0)
    last = (NCHUNK - 1) % 2
    pltpu.make_async_copy(buf.at[last], out_at(NCHUNK - 1),
                          semw.at[last]).wait()


def _sc_gather(ids, t2):
    mesh = plsc.VectorSubcoreMesh(core_axis_name="c", subcore_axis_name="s")
    f = functools.partial(
        pl.kernel,
        mesh=mesh,
        out_type=jax.ShapeDtypeStruct((2 * TOKENS, D), jnp.float32),
        scratch_types=[
            pltpu.VMEM((TPW,), jnp.int32),
            pltpu.VMEM((2 * TPW,), jnp.int32),
            pltpu.VMEM((2, ROWS, D), jnp.float32),
            pltpu.SemaphoreType.DMA((2,)),
            pltpu.SemaphoreType.DMA((2,)),
        ],
    )(_sc_gather_body)
    return f(ids, t2)


def kernel(input_ids, weight, comp_map):
    ids = input_ids.reshape(-1)
    t2 = _build_t2(weight, comp_map)
    out = _sc_gather(ids, t2)
    return out.reshape(input_ids.shape[0], input_ids.shape[1], 2 * D)


# speedup vs baseline: 7.0450x; 7.0450x over previous
import functools

import jax
import jax.numpy as jnp
from jax import lax
from jax.experimental import pallas as pl
from jax.experimental.pallas import tpu as pltpu
from jax.experimental.pallas import tpu_sc as plsc

VOCAB = 16
D = 1024
TOKENS = 4 * 8192

_info = plsc.get_sparse_core_info()
NC, NS = _info.num_cores, _info.num_subcores
NW = NC * NS
TPW = TOKENS // NW
DEPTH = 8


def _build_trc_body(w_ref, cm_ref, trc_ref):
    w = w_ref[...]
    cm = cm_ref[...]
    onehot = (cm == lax.broadcasted_iota(jnp.int32, (VOCAB, VOCAB), 1))
    sel = jax.lax.dot(onehot.astype(jnp.float32), w,
                      precision=jax.lax.Precision.HIGHEST)
    revp = (lax.broadcasted_iota(jnp.int32, (D, D), 0)
            + lax.broadcasted_iota(jnp.int32, (D, D), 1)) == (D - 1)
    trc_ref[...] = jax.lax.dot(sel, revp.astype(jnp.float32),
                               precision=jax.lax.Precision.HIGHEST)


def _build_trc(weight, comp_map):
    return pl.pallas_call(
        _build_trc_body,
        out_shape=jax.ShapeDtypeStruct((VOCAB, D), jnp.float32),
    )(weight, comp_map.reshape(VOCAB, 1))


def _sc_write_body(ids_hbm, w_hbm, trc_hbm, out_hbm, ids_sm, ids_v, tab_v,
                   semw):
    sid = lax.axis_index("s")
    wid = sid * NC + lax.axis_index("c")
    base = wid * TPW

    pltpu.sync_copy(w_hbm, tab_v.at[pl.ds(0, VOCAB), pl.ds(0, D)])
    pltpu.sync_copy(trc_hbm, tab_v.at[pl.ds(0, VOCAB), pl.ds(D, D)])
    pltpu.sync_copy(ids_hbm.at[pl.ds(base, TPW)], ids_v.at[sid])
    pltpu.sync_copy(ids_v.at[sid], ids_sm)

    def step(t, carry):
        pltpu.async_copy(tab_v.at[ids_sm[t]], out_hbm.at[base + t], semw)

        @pl.when(t >= DEPTH)
        def _():
            pltpu.make_async_copy(tab_v.at[0], out_hbm.at[base], semw).wait()

        return carry

    lax.fori_loop(0, TPW, step, 0)

    def drain(j, carry):
        pltpu.make_async_copy(tab_v.at[0], out_hbm.at[base], semw).wait()
        return carry

    lax.fori_loop(0, DEPTH, drain, 0)


def _sc_write(ids, weight, trc):
    mesh = plsc.VectorSubcoreMesh(core_axis_name="c", subcore_axis_name="s")
    f = functools.partial(
        pl.kernel,
        mesh=mesh,
        out_type=jax.ShapeDtypeStruct((TOKENS, 2 * D), jnp.float32),
        scratch_types=[
            pltpu.SMEM((TPW,), jnp.int32),
            pltpu.VMEM_SHARED((NS, TPW), jnp.int32),
            pltpu.VMEM((VOCAB, 2 * D), jnp.float32),
            pltpu.SemaphoreType.DMA,
        ],
    )(_sc_write_body)
    return f(ids, weight, trc)


def kernel(input_ids, weight, comp_map):
    ids = input_ids.reshape(-1)
    trc = _build_trc(weight, comp_map)
    out = _sc_write(ids, weight, trc)
    return out.reshape(input_ids.shape[0], input_ids.shape[1], 2 * D)
